# Initial kernel scaffold; baseline (speedup 1.0000x reference)
import jax
import jax.numpy as jnp
from jax.experimental import pallas as pl


def kernel(user_table, item_table, beta_uD, beta_iD, ii_constraint, users, pos_items, neg_items, ii_neighbor):
    raise NotImplementedError("write your pallas kernel here")


# speedup vs baseline: 2.4114x; 2.4114x over previous
import functools

import jax
import jax.numpy as jnp
from jax import lax
from jax.experimental import pallas as pl
from jax.experimental.pallas import tpu as pltpu
from jax.experimental.pallas import tpu_sc as plsc

_W1 = 1e-06
_W2 = 1.0
_W3 = 1.0
_W4 = 1e-06
_NEG_WEIGHT = 50.0
_GAMMA = 1e-04
_LAMBDA = 2.75

_D = 64
_NNEG = 50
_NNEG_P = 64
_K = 10
_K_P = 16
_NC = 2
_NS = 16
_NW = _NC * _NS
_G = 4


def _merge16(vs, lane):
    for s in range(4):
        sh = 1 << s
        bit = (lane & sh) == 0
        nxt = []
        for m in range(0, len(vs), 2):
            a, b = vs[m], vs[m + 1]
            t = jnp.where(bit, a, b)
            q = jnp.where(bit, b, a)
            nxt.append(t + q[lane ^ sh])
        vs = nxt
    return vs[0]


def _sc_scores(user_table, item_table, bu_t, bi_t, iic_p,
               users, pos_items, neg_items, iin_p):
    B = users.shape[0]
    PB = B // _NW
    NG = PB // _G

    mesh = plsc.VectorSubcoreMesh(core_axis_name="c", subcore_axis_name="s")

    out_type = [
        jax.ShapeDtypeStruct((B,), jnp.float32),
        jax.ShapeDtypeStruct((B, _NNEG_P), jnp.float32),
        jax.ShapeDtypeStruct((B, _K_P), jnp.float32),
        jax.ShapeDtypeStruct((B, _K_P), jnp.float32),
        jax.ShapeDtypeStruct((B, _K_P), jnp.float32),
        jax.ShapeDtypeStruct((B, _NNEG_P), jnp.float32),
        jax.ShapeDtypeStruct((B, _K_P), jnp.float32),
    ]
    scratch_types = [
        pltpu.VMEM((PB,), jnp.int32),
        pltpu.VMEM((PB,), jnp.int32),
        pltpu.VMEM((PB, _NNEG), jnp.int32),
        pltpu.VMEM((PB, _K_P), jnp.int32),
        pltpu.VMEM((PB, _D), jnp.float32),
        pltpu.VMEM((PB, _D), jnp.float32),
        pltpu.VMEM((PB, _K_P), jnp.float32),
        pltpu.VMEM((PB, _K_P), jnp.float32),
        pltpu.VMEM((PB, _K_P), jnp.float32),
        pltpu.VMEM((_G, _NNEG, _D), jnp.float32),
        pltpu.VMEM((_G, _K_P, _D), jnp.float32),
        pltpu.VMEM((_G, _NNEG, _K_P), jnp.float32),
        pltpu.VMEM((PB,), jnp.float32),
        pltpu.VMEM((PB, _NNEG_P), jnp.float32),
        pltpu.VMEM((PB, _K_P), jnp.float32),
        pltpu.VMEM((PB, _NNEG_P), jnp.float32),
        pltpu.SemaphoreType.DMA,
    ]

    @functools.partial(
        pl.kernel, out_type=out_type, mesh=mesh, scratch_types=scratch_types,
        compiler_params=pltpu.CompilerParams(use_tc_tiling_on_sc=False))
    def body(ut, it, bu2, bi2, iic, users_h, pos_h, neg_h, iin,
             ps_o, ns_o, ss_o, bu_o, bi_o, bn_o, sim_o,
             users_v, pos_v, neg_v, nbr_v, ue_v, pe_v, bu_v, bi_v, sim_v,
             ne_buf, nb_buf, bnb, ps_v, ns_v, ss_v, bn_v, sem):
        wid = lax.axis_index("s") * _NC + lax.axis_index("c")
        base = wid * PB
        lane = lax.iota(jnp.int32, 16)
        zeros = jnp.zeros((16,), jnp.float32)

        pltpu.sync_copy(users_h.at[pl.ds(base, PB)], users_v)
        pltpu.sync_copy(pos_h.at[pl.ds(base, PB)], pos_v)
        pltpu.sync_copy(neg_h.at[pl.ds(base, PB)], neg_v)

        cps = [
            pltpu.async_copy(ut.at[users_v], ue_v, sem),
            pltpu.async_copy(it.at[pos_v], pe_v, sem),
            pltpu.async_copy(bu2.at[users_v], bu_v, sem),
            pltpu.async_copy(bi2.at[pos_v], bi_v, sem),
            pltpu.async_copy(iic.at[pos_v], sim_v, sem),
            pltpu.async_copy(iin.at[pos_v], nbr_v, sem),
        ]
        for c in cps:
            c.wait()

        def pos_body(p, carry):
            vs = []
            for r in range(16):
                elem = p * 16 + r
                part = ue_v[elem, pl.ds(0, 16)] * pe_v[elem, pl.ds(0, 16)]
                for t in range(1, 4):
                    part = part + (ue_v[elem, pl.ds(t * 16, 16)]
                                   * pe_v[elem, pl.ds(t * 16, 16)])
                vs.append(part)
            ps_v[pl.ds(p * 16, 16)] = _merge16(vs, lane)
            return carry
        lax.fori_loop(0, PB // 16, pos_body, 0)

        def group_body(g, carry):
            e0 = g * _G
            dmas = []
            for e in range(_G):
                elem = e0 + e
                dmas.append(pltpu.async_copy(
                    it.at[neg_v.at[elem]], ne_buf.at[e], sem))
                dmas.append(pltpu.async_copy(
                    it.at[nbr_v.at[elem]], nb_buf.at[e], sem))
                dmas.append(pltpu.async_copy(
                    bi2.at[neg_v.at[elem]], bnb.at[e], sem))
            for c in dmas:
                c.wait()

            def elem_body(e, c2):
                elem = e0 + e
                u = [ue_v[elem, pl.ds(t * 16, 16)] for t in range(4)]
                for q in range(_NNEG_P // 16):
                    vs, bs = [], []
                    for r in range(16):
                        row = q * 16 + r
                        if row < _NNEG:
                            part = u[0] * ne_buf[e, row, pl.ds(0, 16)]
                            for t in range(1, 4):
                                part = part + u[t] * ne_buf[e, row,
                                                            pl.ds(t * 16, 16)]
                            vs.append(part)
                            bs.append(bnb[e, row, pl.ds(0, 16)])
                        else:
                            vs.append(zeros)
                            bs.append(zeros)
                    ns_v[elem, pl.ds(q * 16, 16)] = _merge16(vs, lane)
                    bn_v[elem, pl.ds(q * 16, 16)] = _merge16(bs, lane)
                vs = []
                for r in range(_K_P):
                    part = u[0] * nb_buf[e, r, pl.ds(0, 16)]
                    for t in range(1, 4):
                        part = part + u[t] * nb_buf[e, r, pl.ds(t * 16, 16)]
                    vs.append(part)
                ss_v[elem, pl.ds(0, 16)] = _merge16(vs, lane)
                return c2
            lax.fori_loop(0, _G, elem_body, 0)
            return carry

        lax.fori_loop(0, NG, group_body, 0)

        pltpu.sync_copy(ps_v, ps_o.at[pl.ds(base, PB)])
        pltpu.sync_copy(ns_v, ns_o.at[pl.ds(base, PB)])
        pltpu.sync_copy(ss_v, ss_o.at[pl.ds(base, PB)])
        pltpu.sync_copy(bu_v, bu_o.at[pl.ds(base, PB)])
        pltpu.sync_copy(bi_v, bi_o.at[pl.ds(base, PB)])
        pltpu.sync_copy(bn_v, bn_o.at[pl.ds(base, PB)])
        pltpu.sync_copy(sim_v, sim_o.at[pl.ds(base, PB)])

    return body(user_table, item_table, bu_t, bi_t, iic_p,
                users, pos_items, neg_items, iin_p)


def _softplus(x):
    return jnp.maximum(x, 0.0) + jnp.log1p(jnp.exp(-jnp.abs(x)))


def _tc_body(nsteps, ut_ref, it_ref, ps_ref, ns_ref, ss_ref, bu_ref, bi_ref,
             bn_ref, sim_ref, out_ref, acc_ref):
    i = pl.program_id(0)

    @pl.when(i == 0)
    def _init():
        acc_ref[0, 0] = 0.0

    x = ut_ref[...]
    y = it_ref[...]
    acc_ref[0, 0] = acc_ref[0, 0] + jnp.sum(x * x) + jnp.sum(y * y)

    @pl.when(i == nsteps - 1)
    def _final():
        B = ps_ref.shape[0]
        bu = bu_ref[...][:, 0:1]
        bi = bi_ref[...][:, 0:1]
        ps = ps_ref[...]
        pos_w = _W1 + _W2 * bu * bi
        pos_loss = jnp.sum(pos_w * _softplus(-ps))

        ns = ns_ref[...]
        bn = bn_ref[...]
        ncol = lax.broadcasted_iota(jnp.int32, (B, _NNEG_P), 1)
        neg_w = _W3 + _W4 * bu * bn
        neg_terms = jnp.where(ncol < _NNEG, neg_w * _softplus(ns), 0.0)
        neg_loss = jnp.sum(neg_terms) * (_NEG_WEIGHT / _NNEG)

        ss = ss_ref[...]
        sim = sim_ref[...]
        kcol = lax.broadcasted_iota(jnp.int32, (B, _K_P), 1)
        i_terms = jnp.where(kcol < _K, sim * _softplus(-ss), 0.0)
        loss_i = jnp.sum(i_terms)

        norm = 0.5 * acc_ref[0, 0]
        total = pos_loss + neg_loss + _GAMMA * norm + _LAMBDA * loss_i
        out_ref[...] = jnp.reshape(total, (1, 1))


def _tc_loss(user_table, item_table, ps, ns, ss, bu, bi, bn, sim):
    rows = user_table.shape[0]
    block_rows = 10000
    nsteps = rows // block_rows
    B = ps.shape[0]

    const = lambda i: (0, 0)
    return pl.pallas_call(
        functools.partial(_tc_body, nsteps),
        grid=(nsteps,),
        in_specs=[
            pl.BlockSpec((block_rows, _D), lambda i: (i, 0)),
            pl.BlockSpec((block_rows, _D), lambda i: (i, 0)),
            pl.BlockSpec((B, 1), const),
            pl.BlockSpec((B, _NNEG_P), const),
            pl.BlockSpec((B, _K_P), const),
            pl.BlockSpec((B, _K_P), const),
            pl.BlockSpec((B, _K_P), const),
            pl.BlockSpec((B, _NNEG_P), const),
            pl.BlockSpec((B, _K_P), const),
        ],
        out_specs=pl.BlockSpec((1, 1), const),
        out_shape=jax.ShapeDtypeStruct((1, 1), jnp.float32),
        scratch_shapes=[pltpu.SMEM((1, 1), jnp.float32)],
    )(user_table, item_table, ps, ns, ss, bu, bi, bn, sim)


def kernel(user_table, item_table, beta_uD, beta_iD, ii_constraint,
           users, pos_items, neg_items, ii_neighbor):
    bu_t = jnp.pad(beta_uD[:, None], ((0, 0), (0, _K_P - 1)))
    bi_t = jnp.pad(beta_iD[:, None], ((0, 0), (0, _K_P - 1)))
    iic_p = jnp.pad(ii_constraint, ((0, 0), (0, _K_P - _K)))
    iin_p = jnp.pad(ii_neighbor, ((0, 0), (0, _K_P - _K)))

    ps, ns, ss, bu, bi, bn, sim = _sc_scores(
        user_table, item_table, bu_t, bi_t, iic_p,
        users, pos_items, neg_items, iin_p)

    B = users.shape[0]
    out = _tc_loss(user_table, item_table, ps.reshape(B, 1), ns, ss,
                   bu, bi, bn, sim)
    return out[0, 0]
